# Initial kernel scaffold; baseline (speedup 1.0000x reference)
import jax
import jax.numpy as jnp
from jax.experimental import pallas as pl


def kernel(mem_img, mem_gps, mem_coords, img_emb, gps_emb, gps_coords, ptr):
    raise NotImplementedError("write your pallas kernel here")


# speedup vs baseline: 3.1649x; 3.1649x over previous
import jax
import jax.numpy as jnp
from jax.experimental import pallas as pl
from jax.experimental.pallas import tpu as pltpu

_M = 65536
_B = 4096
_D = 512
_C = 2 * _D + 2
_R = 512
_E = _B + 2 * _R + 8


def _body(ptr_ref, mi_ref, mg_ref, mc_ref, ie_ref, ge_ref, ce_ref, out_ref):
    i = pl.program_id(0)
    ptr = ptr_ref[0]
    r0 = i * _R
    d = r0 - ptr
    j0 = jnp.where(d < 0, d + _M, d)

    k = jax.lax.broadcasted_iota(jnp.int32, (_R, 1), 0)
    jv = j0 + k
    jm = jnp.where(jv >= _M, jv - _M, jv)
    mask = jm < _B

    f = _R + ptr % 8
    start = jnp.where(j0 < _B, f + j0,
                      jnp.where(j0 >= _M - _R, f + j0 - _M, 0))
    start = pl.multiple_of(start, 8)

    out_ref[:, 0:_D] = jnp.where(mask, ie_ref[pl.ds(start, _R), :], mi_ref[...])
    out_ref[:, _D:2 * _D] = jnp.where(mask, ge_ref[pl.ds(start, _R), :], mg_ref[...])
    out_ref[:, 2 * _D:_C] = jnp.where(mask, ce_ref[pl.ds(start, _R), :], mc_ref[...])


def kernel(mem_img, mem_gps, mem_coords, img_emb, gps_emb, gps_coords, ptr):
    ptr32 = jnp.asarray(ptr, jnp.int32) % _M
    pvec = ptr32.reshape((1,))
    f = _R + ptr32 % 8
    zero = jnp.int32(0)

    def stage(x, cols):
        buf = jnp.zeros((_E, cols), jnp.float32)
        return jax.lax.dynamic_update_slice(buf, x, (f, zero))

    img_ext = stage(img_emb, _D)
    gps_ext = stage(gps_emb, _D)
    crd_ext = stage(gps_coords, 2)
    grid_spec = pltpu.PrefetchScalarGridSpec(
        num_scalar_prefetch=1,
        grid=(_M // _R,),
        in_specs=[
            pl.BlockSpec((_R, _D), lambda i, p: (i, 0)),
            pl.BlockSpec((_R, _D), lambda i, p: (i, 0)),
            pl.BlockSpec((_R, 2), lambda i, p: (i, 0)),
            pl.BlockSpec((_E, _D), lambda i, p: (0, 0)),
            pl.BlockSpec((_E, _D), lambda i, p: (0, 0)),
            pl.BlockSpec((_E, 2), lambda i, p: (0, 0)),
        ],
        out_specs=pl.BlockSpec((_R, _C), lambda i, p: (i, 0)),
    )
    return pl.pallas_call(
        _body,
        grid_spec=grid_spec,
        out_shape=jax.ShapeDtypeStruct((_M, _C), jnp.float32),
    )(pvec, mem_img, mem_gps, mem_coords, img_ext, gps_ext, crd_ext)


# speedup vs baseline: 3.5742x; 1.1293x over previous
import jax
import jax.numpy as jnp
from jax.experimental import pallas as pl
from jax.experimental.pallas import tpu as pltpu

_M = 65536
_B = 4096
_D = 512
_C = 2 * _D + 2
_R = 512
_W = _R + 8


def _body(ptr_ref, mi_ref, mg_ref, mc_ref, ie_ref, ge_ref, ce_ref, out_ref):
    i = pl.program_id(0)
    ptr = ptr_ref[0]
    r0 = i * _R
    d = r0 - ptr
    j0 = jnp.where(d < 0, d + _M, d)

    straight = j0 < _B
    wrapped = j0 > _M - _R
    overlap = straight | wrapped

    @pl.when(jnp.logical_not(overlap))
    def _copy():
        out_ref[:, 0:_D] = mi_ref[...]
        out_ref[:, _D:2 * _D] = mg_ref[...]
        out_ref[:, 2 * _D:_C] = mc_ref[...]

    @pl.when(overlap)
    def _select():
        k = jax.lax.broadcasted_iota(jnp.int32, (_R, 1), 0)
        jv = j0 + k
        jm = jnp.where(jv >= _M, jv - _M, jv)
        mask = jm < _B
        w0 = jnp.where(straight, jnp.minimum((j0 >> 3) << 3, _B - _W), 0)
        w0 = pl.multiple_of(w0, 8)
        delta = jnp.where(straight, j0 - w0, _W - (_M - j0))
        shift = jnp.where(delta == 0, 0, _W - delta)

        def pick(new_ref, mem_ref):
            win = pltpu.roll(new_ref[pl.ds(w0, _W), :], shift, 0)
            return jnp.where(mask, win[0:_R], mem_ref[...])

        out_ref[:, 0:_D] = pick(ie_ref, mi_ref)
        out_ref[:, _D:2 * _D] = pick(ge_ref, mg_ref)
        out_ref[:, 2 * _D:_C] = pick(ce_ref, mc_ref)


def kernel(mem_img, mem_gps, mem_coords, img_emb, gps_emb, gps_coords, ptr):
    pvec = (jnp.asarray(ptr, jnp.int32) % _M).reshape((1,))
    grid_spec = pltpu.PrefetchScalarGridSpec(
        num_scalar_prefetch=1,
        grid=(_M // _R,),
        in_specs=[
            pl.BlockSpec((_R, _D), lambda i, p: (i, 0)),
            pl.BlockSpec((_R, _D), lambda i, p: (i, 0)),
            pl.BlockSpec((_R, 2), lambda i, p: (i, 0)),
            pl.BlockSpec((_B, _D), lambda i, p: (0, 0)),
            pl.BlockSpec((_B, _D), lambda i, p: (0, 0)),
            pl.BlockSpec((_B, 2), lambda i, p: (0, 0)),
        ],
        out_specs=pl.BlockSpec((_R, _C), lambda i, p: (i, 0)),
    )
    return pl.pallas_call(
        _body,
        grid_spec=grid_spec,
        out_shape=jax.ShapeDtypeStruct((_M, _C), jnp.float32),
    )(pvec, mem_img, mem_gps, mem_coords, img_emb, gps_emb, gps_coords)


# speedup vs baseline: 3.5950x; 1.0058x over previous
import jax
import jax.numpy as jnp
from jax.experimental import pallas as pl
from jax.experimental.pallas import tpu as pltpu

_M = 65536
_B = 4096
_D = 512
_C = 2 * _D + 2
_R = 512
_W = _R + 8


def _body(ptr_ref, mi_ref, mg_ref, mc_ref, ie_ref, ge_ref, ce_ref, out_ref):
    i = pl.program_id(0)
    ptr = ptr_ref[0]
    r0 = i * _R
    d = r0 - ptr
    j0 = jnp.where(d < 0, d + _M, d)

    straight = j0 < _B
    wrapped = j0 > _M - _R
    overlap = straight | wrapped

    @pl.when(jnp.logical_not(overlap))
    def _copy():
        out_ref[:, 0:_D] = mi_ref[...]
        out_ref[:, _D:2 * _D] = mg_ref[...]
        out_ref[:, 2 * _D:_C] = mc_ref[...]

    @pl.when(overlap)
    def _select():
        k = jax.lax.broadcasted_iota(jnp.int32, (_R, 1), 0)
        jv = j0 + k
        jm = jnp.where(jv >= _M, jv - _M, jv)
        mask = jm < _B
        w0 = jnp.where(straight, jnp.minimum((j0 >> 3) << 3, _B - _W), 0)
        w0 = pl.multiple_of(w0, 8)
        delta = jnp.where(straight, j0 - w0, _W - (_M - j0))
        shift = jnp.where(delta == 0, 0, _W - delta)

        def pick(new_ref, mem_ref):
            win = pltpu.roll(new_ref[pl.ds(w0, _W), :], shift, 0)
            return jnp.where(mask, win[0:_R], mem_ref[...])

        out_ref[:, 0:_D] = pick(ie_ref, mi_ref)
        out_ref[:, _D:2 * _D] = pick(ge_ref, mg_ref)
        out_ref[:, 2 * _D:_C] = pick(ce_ref, mc_ref)


def kernel(mem_img, mem_gps, mem_coords, img_emb, gps_emb, gps_coords, ptr):
    pvec = (jnp.asarray(ptr, jnp.int32) % _M).reshape((1,))
    nblk = _M // _R

    def mem_idx(i, p):
        j0 = jax.lax.rem(i * _R - p[0] + _M, _M)
        covered = j0 <= _B - _R
        prev = jax.lax.rem((p[0] + _R - 1) // _R + nblk - 1, nblk)
        return (jnp.where(covered, prev, i), 0)

    grid_spec = pltpu.PrefetchScalarGridSpec(
        num_scalar_prefetch=1,
        grid=(nblk,),
        in_specs=[
            pl.BlockSpec((_R, _D), mem_idx),
            pl.BlockSpec((_R, _D), mem_idx),
            pl.BlockSpec((_R, 2), mem_idx),
            pl.BlockSpec((_B, _D), lambda i, p: (0, 0)),
            pl.BlockSpec((_B, _D), lambda i, p: (0, 0)),
            pl.BlockSpec((_B, 2), lambda i, p: (0, 0)),
        ],
        out_specs=pl.BlockSpec((_R, _C), lambda i, p: (i, 0)),
    )
    return pl.pallas_call(
        _body,
        grid_spec=grid_spec,
        out_shape=jax.ShapeDtypeStruct((_M, _C), jnp.float32),
    )(pvec, mem_img, mem_gps, mem_coords, img_emb, gps_emb, gps_coords)


# speedup vs baseline: 3.6577x; 1.0174x over previous
import jax
import jax.numpy as jnp
from jax.experimental import pallas as pl
from jax.experimental.pallas import tpu as pltpu

_M = 65536
_B = 4096
_D = 512
_C = 2 * _D + 2
_R = 1024
_W = _R + 8


def _body(ptr_ref, mi_ref, mg_ref, mc_ref, ie_ref, ge_ref, ce_ref, out_ref):
    i = pl.program_id(0)
    ptr = ptr_ref[0]
    r0 = i * _R
    d = r0 - ptr
    j0 = jnp.where(d < 0, d + _M, d)

    straight = j0 < _B
    wrapped = j0 > _M - _R
    overlap = straight | wrapped

    @pl.when(jnp.logical_not(overlap))
    def _copy():
        out_ref[:, 0:_D] = mi_ref[...]
        out_ref[:, _D:2 * _D] = mg_ref[...]
        out_ref[:, 2 * _D:_C] = mc_ref[...]

    @pl.when(overlap)
    def _select():
        k = jax.lax.broadcasted_iota(jnp.int32, (_R, 1), 0)
        jv = j0 + k
        jm = jnp.where(jv >= _M, jv - _M, jv)
        mask = jm < _B
        w0 = jnp.where(straight, jnp.minimum((j0 >> 3) << 3, _B - _W), 0)
        w0 = pl.multiple_of(w0, 8)
        delta = jnp.where(straight, j0 - w0, _W - (_M - j0))
        shift = jnp.where(delta == 0, 0, _W - delta)

        def pick(new_ref, mem_ref):
            win = pltpu.roll(new_ref[pl.ds(w0, _W), :], shift, 0)
            return jnp.where(mask, win[0:_R], mem_ref[...])

        out_ref[:, 0:_D] = pick(ie_ref, mi_ref)
        out_ref[:, _D:2 * _D] = pick(ge_ref, mg_ref)
        out_ref[:, 2 * _D:_C] = pick(ce_ref, mc_ref)


def kernel(mem_img, mem_gps, mem_coords, img_emb, gps_emb, gps_coords, ptr):
    pvec = (jnp.asarray(ptr, jnp.int32) % _M).reshape((1,))
    nblk = _M // _R

    def mem_idx(i, p):
        j0 = jax.lax.rem(i * _R - p[0] + _M, _M)
        covered = j0 <= _B - _R
        prev = jax.lax.rem((p[0] + _R - 1) // _R + nblk - 1, nblk)
        return (jnp.where(covered, prev, i), 0)

    grid_spec = pltpu.PrefetchScalarGridSpec(
        num_scalar_prefetch=1,
        grid=(nblk,),
        in_specs=[
            pl.BlockSpec((_R, _D), mem_idx),
            pl.BlockSpec((_R, _D), mem_idx),
            pl.BlockSpec((_R, 2), mem_idx),
            pl.BlockSpec((_B, _D), lambda i, p: (0, 0)),
            pl.BlockSpec((_B, _D), lambda i, p: (0, 0)),
            pl.BlockSpec((_B, 2), lambda i, p: (0, 0)),
        ],
        out_specs=pl.BlockSpec((_R, _C), lambda i, p: (i, 0)),
    )
    return pl.pallas_call(
        _body,
        grid_spec=grid_spec,
        out_shape=jax.ShapeDtypeStruct((_M, _C), jnp.float32),
    )(pvec, mem_img, mem_gps, mem_coords, img_emb, gps_emb, gps_coords)
